# baseline (speedup 1.0000x reference)
import functools

import jax
import jax.numpy as jnp
from jax import lax
from jax.experimental import pallas as pl
from jax.experimental.pallas import tpu as pltpu
from jax.experimental.pallas import tpu_sc as plsc

N = 4096
NC = 2
NS = 16
NW = NC * NS
L = 16
NSC = 512
ROWS = NSC // NW
CHUNKS = N // L
TC_BR = 256


def _body(e_hbm, t_hbm, est_hbm, out_hbm, e_v, t_v, est_v, part_v):
    cid = lax.axis_index("c")
    sid = lax.axis_index("s")
    wid = sid * NC + cid

    pltpu.sync_copy(e_hbm, e_v)
    pltpu.sync_copy(t_hbm, t_v)
    pltpu.sync_copy(est_hbm, est_v)

    def _exp_chunk(i, carry):
        t_v[pl.ds(i * L, L)] = jnp.exp(t_v[pl.ds(i * L, L)])
        return carry

    lax.fori_loop(0, CHUNKS, _exp_chunk, 0)

    zeros = jnp.zeros((L,), jnp.float32)
    ones = jnp.ones((L,), jnp.float32)
    izeros = jnp.zeros((L,), jnp.int32)

    part_v[0, :] = zeros
    part_v[1, :] = zeros
    part_v[2, :] = zeros

    def _row_chunk(rc, carry0):
        base = wid * ROWS + rc * L
        rowt = t_v[pl.ds(base, L)]
        rowe = e_v[pl.ds(base, L)]
        rowest = est_v[pl.ds(base, L)]
        for r in range(L):
            ta_s = rowt[r]
            esta_s = rowest[r]
            ea_s = rowe[r]

            @pl.when(ea_s > 0.5)
            def _do_row(ta_s=ta_s, esta_s=esta_s):
                ta = jnp.full((L,), ta_s, jnp.float32)
                esta = jnp.full((L,), esta_s, jnp.float32)

                def _chunk(c, carry):
                    acc_t, acc_c, acc_e, acc_q = carry
                    off = c * L
                    tb = t_v[pl.ds(off, L)]
                    estb = est_v[pl.ds(off, L)]
                    m_gt = tb > ta
                    m_con = estb < esta
                    m_tie = jnp.abs(estb - esta) <= 1e-8
                    acc_t = acc_t + jnp.where(m_gt, ones, zeros)
                    acc_c = acc_c + jnp.where(m_gt & m_con, ones, zeros)
                    acc_e = acc_e + jnp.where(m_gt & m_tie, ones, zeros)
                    acc_q = acc_q + jnp.where(tb == ta, ones, zeros)
                    return acc_t, acc_c, acc_e, acc_q

                acc_t, acc_c, acc_e, acc_q = lax.fori_loop(
                    0, CHUNKS, _chunk, (zeros, zeros, zeros, zeros))
                part_v[0, :] = part_v[0, :] + acc_t
                part_v[1, :] = part_v[1, :] + acc_c
                part_v[2, :] = part_v[2, :] + acc_e

                q = acc_q[0]
                for lane in range(1, L):
                    q = q + acc_q[lane]

                @pl.when(q > 1.5)
                def _tied_fix():
                    def _chunk2(c, carry):
                        a_t, a_c, a_e = carry
                        off = c * L
                        tb = t_v[pl.ds(off, L)]
                        eb = e_v[pl.ds(off, L)]
                        estb = est_v[pl.ds(off, L)]
                        m_c = (tb == ta) & (eb < 0.5)
                        m_con = estb < esta
                        m_tie = jnp.abs(estb - esta) <= 1e-8
                        a_t = a_t + jnp.where(m_c, ones, zeros)
                        a_c = a_c + jnp.where(m_c & m_con, ones, zeros)
                        a_e = a_e + jnp.where(m_c & m_tie, ones, zeros)
                        return a_t, a_c, a_e

                    f_t, f_c, f_e = lax.fori_loop(
                        0, CHUNKS, _chunk2, (zeros, zeros, zeros))
                    part_v[0, :] = part_v[0, :] + f_t
                    part_v[1, :] = part_v[1, :] + f_c
                    part_v[2, :] = part_v[2, :] + f_e

        return carry0

    lax.fori_loop(0, ROWS // L, _row_chunk, 0)
    pltpu.sync_copy(part_v, out_hbm.at[wid])


@jax.jit
def _sc_counts(e, t, est):
    mesh = plsc.VectorSubcoreMesh(core_axis_name="c", subcore_axis_name="s")
    f = functools.partial(
        pl.kernel,
        mesh=mesh,
        out_type=jax.ShapeDtypeStruct((NW, 3, L), jnp.float32),
        scratch_types=[
            pltpu.VMEM((N,), jnp.float32),
            pltpu.VMEM((N,), jnp.float32),
            pltpu.VMEM((N,), jnp.float32),
            pltpu.VMEM((3, L), jnp.float32),
        ],
    )(_body)
    return f(e, t, est)


def _tc_body(tr_ref, er_ref, estr_ref, tc_ref, ec_ref, estc_ref, out_ref):
    i = pl.program_id(0)
    ta = jnp.exp(tr_ref[...])
    ea = er_ref[...]
    esta = estr_ref[...]
    tb = jnp.exp(tc_ref[...])
    eb = ec_ref[...]
    estb = estc_ref[...]
    comp = (ea > 0.5) & ((tb > ta) | ((tb == ta) & (eb < 0.5)))
    comp_f = jnp.where(comp, 1.0, 0.0)
    con_f = jnp.where(comp & (estb < esta), 1.0, 0.0)
    tie_f = jnp.where(comp & (jnp.abs(estb - esta) <= 1e-8), 1.0, 0.0)

    @pl.when(i == 0)
    def _init():
        out_ref[0] = 0.0
        out_ref[1] = 0.0
        out_ref[2] = 0.0

    out_ref[0] += jnp.sum(comp_f)
    out_ref[1] += jnp.sum(con_f)
    out_ref[2] += jnp.sum(tie_f)


@jax.jit
def _tc_counts(t_rows, e_rows, est_rows, t, e, est):
    n_rows = t_rows.shape[0]
    grid = n_rows // TC_BR
    row_spec = pl.BlockSpec((TC_BR, 1), lambda i: (i, 0))
    col_spec = pl.BlockSpec((1, N), lambda i: (0, 0))
    return pl.pallas_call(
        _tc_body,
        grid=(grid,),
        in_specs=[row_spec, row_spec, row_spec, col_spec, col_spec, col_spec],
        out_specs=pl.BlockSpec(memory_space=pltpu.SMEM),
        out_shape=jax.ShapeDtypeStruct((3,), jnp.float32),
    )(t_rows, e_rows, est_rows, t, e, est)


def kernel(event_indicator, event_time, estimate):
    e = jnp.reshape(event_indicator, (-1,)).astype(jnp.float32)
    t = jnp.reshape(event_time, (-1,)).astype(jnp.float32)
    est = jnp.reshape(estimate, (-1,)).astype(jnp.float32)
    parts = _sc_counts(e, t, est)
    tc = _tc_counts(
        t[NSC:].reshape(-1, 1), e[NSC:].reshape(-1, 1),
        est[NSC:].reshape(-1, 1), t.reshape(1, -1), e.reshape(1, -1),
        est.reshape(1, -1))
    total = jnp.sum(parts[:, 0, :]) + tc[0]
    con = jnp.sum(parts[:, 1, :]) + tc[1]
    tie = jnp.sum(parts[:, 2, :]) + tc[2]
    disc = total - con - tie
    loss = (disc + 0.5 * tie) / (disc + con + tie + 1e-07)
    return 1.0 - loss


# speedup vs baseline: 1.3690x; 1.3690x over previous
import functools

import jax
import jax.numpy as jnp
from jax import lax
from jax.experimental import pallas as pl
from jax.experimental.pallas import tpu as pltpu
from jax.experimental.pallas import tpu_sc as plsc

N = 4096
NC = 2
NS = 16
NW = NC * NS
L = 16
NSC = 1536
ROWS = NSC // NW
CHUNKS = N // L
TC_BR = 256


def _sc_body(e_hbm, t_hbm, est_hbm, out_hbm, e_v, t_v, est_v, k_v, part_v):
    cid = lax.axis_index("c")
    sid = lax.axis_index("s")
    wid = sid * NC + cid

    pltpu.sync_copy(e_hbm, e_v)
    pltpu.sync_copy(t_hbm, t_v)
    pltpu.sync_copy(est_hbm, est_v)

    def _key_chunk(i, carry):
        sl = pl.ds(i * L, L)
        kb = lax.bitcast_convert_type(jnp.exp(t_v[sl]), jnp.int32)
        k_v[sl] = kb + (1 - e_v[sl])
        return carry

    lax.fori_loop(0, CHUNKS, _key_chunk, 0)

    zeros = jnp.zeros((L,), jnp.float32)
    ones = jnp.ones((L,), jnp.float32)

    part_v[0, :] = zeros
    part_v[1, :] = zeros
    part_v[2, :] = zeros

    def _row_chunk(rc, carry0):
        base = wid * ROWS + rc * L
        rowk = k_v[pl.ds(base, L)]
        rowe = e_v[pl.ds(base, L)]
        rowest = est_v[pl.ds(base, L)]
        for r in range(L):
            ka_s = rowk[r]
            esta_s = rowest[r]
            ea_s = rowe[r]

            @pl.when(ea_s > 0)
            def _do_row(ka_s=ka_s, esta_s=esta_s):
                ka = jnp.full((L,), ka_s, jnp.int32)
                esta = jnp.full((L,), esta_s, jnp.float32)

                def _chunk(c, carry):
                    acc_t, acc_c, acc_e = carry
                    off = c * L
                    kb = k_v[pl.ds(off, L)]
                    estb = est_v[pl.ds(off, L)]
                    m1 = kb > ka
                    m_con = estb < esta
                    m_tie = jnp.abs(estb - esta) <= 1e-8
                    acc_t = acc_t + jnp.where(m1, ones, zeros)
                    acc_c = acc_c + jnp.where(m1 & m_con, ones, zeros)
                    acc_e = acc_e + jnp.where(m1 & m_tie, ones, zeros)
                    return acc_t, acc_c, acc_e

                acc_t, acc_c, acc_e = lax.fori_loop(
                    0, CHUNKS, _chunk, (zeros, zeros, zeros))
                part_v[0, :] = part_v[0, :] + acc_t
                part_v[1, :] = part_v[1, :] + acc_c
                part_v[2, :] = part_v[2, :] + acc_e

        return carry0

    lax.fori_loop(0, ROWS // L, _row_chunk, 0)
    pltpu.sync_copy(part_v, out_hbm.at[wid])


@jax.jit
def _sc_counts(e, t, est):
    mesh = plsc.VectorSubcoreMesh(core_axis_name="c", subcore_axis_name="s")
    f = functools.partial(
        pl.kernel,
        mesh=mesh,
        out_type=jax.ShapeDtypeStruct((NW, 3, L), jnp.float32),
        scratch_types=[
            pltpu.VMEM((N,), jnp.int32),
            pltpu.VMEM((N,), jnp.float32),
            pltpu.VMEM((N,), jnp.float32),
            pltpu.VMEM((N,), jnp.int32),
            pltpu.VMEM((3, L), jnp.float32),
        ],
    )(_sc_body)
    return f(e, t, est)


def _tc_body(tr_ref, er_ref, estr_ref, tc_ref, ec_ref, estc_ref, out_ref):
    i = pl.program_id(0)
    ka = lax.bitcast_convert_type(jnp.exp(tr_ref[...]), jnp.int32)
    ea = er_ref[...]
    esta = estr_ref[...]
    kb = lax.bitcast_convert_type(jnp.exp(tc_ref[...]), jnp.int32)
    kb = kb + (1 - ec_ref[...])
    estb = estc_ref[...]
    comp = (ea > 0) & (kb > ka)
    comp_f = jnp.where(comp, 1.0, 0.0)
    con_f = jnp.where(comp & (estb < esta), 1.0, 0.0)
    tie_f = jnp.where(comp & (jnp.abs(estb - esta) <= 1e-8), 1.0, 0.0)

    @pl.when(i == 0)
    def _init():
        out_ref[0] = 0.0
        out_ref[1] = 0.0
        out_ref[2] = 0.0

    out_ref[0] += jnp.sum(comp_f)
    out_ref[1] += jnp.sum(con_f)
    out_ref[2] += jnp.sum(tie_f)


@jax.jit
def _tc_counts(t_rows, e_rows, est_rows, t, e, est):
    n_rows = t_rows.shape[0]
    grid = n_rows // TC_BR
    row_spec = pl.BlockSpec((TC_BR, 1), lambda i: (i, 0))
    col_spec = pl.BlockSpec((1, N), lambda i: (0, 0))
    return pl.pallas_call(
        _tc_body,
        grid=(grid,),
        in_specs=[row_spec, row_spec, row_spec, col_spec, col_spec, col_spec],
        out_specs=pl.BlockSpec(memory_space=pltpu.SMEM),
        out_shape=jax.ShapeDtypeStruct((3,), jnp.float32),
    )(t_rows, e_rows, est_rows, t, e, est)


def kernel(event_indicator, event_time, estimate):
    e = jnp.reshape(event_indicator, (-1,)).astype(jnp.int32)
    t = jnp.reshape(event_time, (-1,)).astype(jnp.float32)
    est = jnp.reshape(estimate, (-1,)).astype(jnp.float32)
    parts = _sc_counts(e, t, est)
    tc = _tc_counts(
        t[NSC:].reshape(-1, 1), e[NSC:].reshape(-1, 1),
        est[NSC:].reshape(-1, 1), t.reshape(1, -1), e.reshape(1, -1),
        est.reshape(1, -1))
    total = jnp.sum(parts[:, 0, :]) + tc[0]
    con = jnp.sum(parts[:, 1, :]) + tc[1]
    tie = jnp.sum(parts[:, 2, :]) + tc[2]
    disc = total - con - tie
    loss = (disc + 0.5 * tie) / (disc + con + tie + 1e-07)
    return 1.0 - loss
